# baseline (speedup 1.0000x reference)
import functools

import jax
import jax.numpy as jnp
from jax import lax
from jax.experimental import pallas as pl
from jax.experimental.pallas import tpu as pltpu
from jax.experimental.pallas import tpu_sc as plsc

_D = 64
_SCALE = 8.0
_IDXW = 128
_CHUNK = 1024


@functools.cache
def _build(B, V):
    info = plsc.get_sparse_core_info()
    NC, NS = info.num_cores, info.num_subcores
    NW = NC * NS
    K = _CHUNK // _IDXW
    rows_per_w = B // NW
    idxrows_per_w = rows_per_w // _IDXW
    n_chunks = rows_per_w // _CHUNK
    assert rows_per_w % _CHUNK == 0 and B % (NW * _IDXW) == 0

    mesh = plsc.VectorSubcoreMesh(core_axis_name="c", subcore_axis_name="s")

    @functools.partial(
        pl.kernel,
        mesh=mesh,
        out_type=jax.ShapeDtypeStruct((B, _D), jnp.float32),
        scratch_types=[
            pltpu.VMEM((K, _IDXW), jnp.int32),
            pltpu.VMEM((_CHUNK, _D), jnp.float32),
            pltpu.SemaphoreType.DMA,
        ],
        compiler_params=pltpu.CompilerParams(use_tc_tiling_on_sc=False),
    )
    def body(x_hbm, t_hbm, out_hbm, idx_v, rows_v, sem):
        wid = lax.axis_index("s") * NC + lax.axis_index("c")
        row_base = wid * idxrows_per_w

        def chunk(i, carry):
            roff = row_base + i * K
            pltpu.sync_copy(x_hbm.at[pl.ds(roff, K)], idx_v)
            cps = [
                pltpu.async_copy(
                    t_hbm.at[idx_v.at[j]],
                    rows_v.at[pl.ds(j * _IDXW, _IDXW)],
                    sem,
                )
                for j in range(K)
            ]
            for cp in cps:
                cp.wait()

            RU = 8

            def srow(r, c2):
                for u in range(RU):
                    row = r * RU + u
                    for j4 in range(_D // 16):
                        sl = pl.ds(16 * j4, 16)
                        rows_v[row, sl] = rows_v[row, sl] * _SCALE
                return c2

            lax.fori_loop(0, _CHUNK // RU, srow, 0)
            pltpu.sync_copy(rows_v, out_hbm.at[pl.ds(roff * _IDXW, _CHUNK)])
            return carry

        lax.fori_loop(0, n_chunks, chunk, 0)

    return body


def kernel(x, table):
    x2d = x.reshape(-1, _IDXW).astype(jnp.int32)
    B = x2d.shape[0] * _IDXW
    out = _build(B, table.shape[0])(x2d, table)
    return out.reshape(x.shape + (table.shape[-1],))


# speedup vs baseline: 1.0596x; 1.0596x over previous
import functools

import jax
import jax.numpy as jnp
from jax import lax
from jax.experimental import pallas as pl
from jax.experimental.pallas import tpu as pltpu
from jax.experimental.pallas import tpu_sc as plsc

_D = 64
_SCALE = 8.0
_IDXW = 128
_CHUNK = 512
_NBUF = 3


@functools.cache
def _build(B, V):
    info = plsc.get_sparse_core_info()
    NC, NS = info.num_cores, info.num_subcores
    NW = NC * NS
    K = _CHUNK // _IDXW
    rows_per_w = B // NW
    idxrows_per_w = rows_per_w // _IDXW
    n_chunks = rows_per_w // _CHUNK
    assert rows_per_w % _CHUNK == 0 and B % (NW * _IDXW) == 0
    n_main = ((n_chunks - 1) // _NBUF) * _NBUF
    assert n_main >= _NBUF

    mesh = plsc.VectorSubcoreMesh(core_axis_name="c", subcore_axis_name="s")

    @functools.partial(
        pl.kernel,
        mesh=mesh,
        out_type=jax.ShapeDtypeStruct((B, _D), jnp.float32),
        scratch_types=[
            pltpu.VMEM((idxrows_per_w, _IDXW), jnp.int32),
            pltpu.VMEM((_NBUF, _CHUNK, _D), jnp.float32),
            pltpu.SemaphoreType.DMA((_NBUF,)),
            pltpu.SemaphoreType.DMA((_NBUF,)),
        ],
        compiler_params=pltpu.CompilerParams(use_tc_tiling_on_sc=False),
    )
    def body(x_hbm, t_hbm, out_hbm, idx_v, bufs, sg, so):
        wid = lax.axis_index("s") * NC + lax.axis_index("c")
        row_base = wid * idxrows_per_w

        def fire_gather(s, c):
            for j in range(K):
                pltpu.async_copy(
                    t_hbm.at[idx_v.at[c * K + j]],
                    bufs.at[s, pl.ds(j * _IDXW, _IDXW)],
                    sg.at[s],
                )

        def drain_gather(s):
            pltpu.make_async_copy(
                t_hbm.at[pl.ds(0, _CHUNK)], bufs.at[s], sg.at[s]
            ).wait()

        def scale(s):
            RU = 8

            def srow(r, carry):
                for u in range(RU):
                    row = r * RU + u
                    for j4 in range(_D // 16):
                        sl = pl.ds(16 * j4, 16)
                        bufs[s, row, sl] = bufs[s, row, sl] * _SCALE
                return carry

            lax.fori_loop(0, _CHUNK // RU, srow, 0)

        def fire_out(s, c):
            off = (row_base + c * K) * _IDXW
            pltpu.async_copy(bufs.at[s], out_hbm.at[pl.ds(off, _CHUNK)], so.at[s])

        def wait_out(s):
            pltpu.make_async_copy(
                t_hbm.at[pl.ds(0, _CHUNK)], bufs.at[s], so.at[s]
            ).wait()

        pltpu.sync_copy(x_hbm.at[pl.ds(row_base, idxrows_per_w)], idx_v)
        fire_gather(0, 0)
        fire_gather(1, 1)

        drain_gather(0)
        scale(0)
        fire_gather(2, 2)
        fire_out(0, 0)

        drain_gather(1)
        scale(1)
        wait_out(0)
        fire_gather(0, 3)
        fire_out(1, 1)

        drain_gather(2)
        scale(2)
        wait_out(1)
        fire_gather(1, 4)
        fire_out(2, 2)

        def ring(g, carry):
            c0 = g * _NBUF
            for k in range(_NBUF):
                s = k
                c = c0 + k
                drain_gather(s)
                scale(s)
                wait_out((k + 2) % _NBUF)
                fire_gather((k + 2) % _NBUF, c + 2)
                fire_out(s, c)
            return carry

        lax.fori_loop(1, n_main // _NBUF, ring, 0)

        for c in range(n_main, n_chunks):
            s = c % _NBUF
            drain_gather(s)
            scale(s)
            wait_out((s + 2) % _NBUF)
            fire_out(s, c)

        wait_out((n_chunks - 1) % _NBUF)

    return body


def kernel(x, table):
    x2d = x.reshape(-1, _IDXW).astype(jnp.int32)
    B = x2d.shape[0] * _IDXW
    out = _build(B, table.shape[0])(x2d, table)
    return out.reshape(x.shape + (table.shape[-1],))


# speedup vs baseline: 1.0601x; 1.0005x over previous
import functools

import jax
import jax.numpy as jnp
from jax import lax
from jax.experimental import pallas as pl
from jax.experimental.pallas import tpu as pltpu
from jax.experimental.pallas import tpu_sc as plsc

_D = 64
_SCALE = 8.0
_SENT = 2
_NBUF = 3


@functools.cache
def _build(BATCH, T, V):
    info = plsc.get_sparse_core_info()
    NC, NS = info.num_cores, info.num_subcores
    NW = NC * NS
    sents_per_w = BATCH // NW
    n_chunks = sents_per_w // _SENT
    assert BATCH % NW == 0 and sents_per_w % _SENT == 0
    n_main = ((n_chunks - 4) // _NBUF) * _NBUF
    assert n_main >= _NBUF and n_chunks - n_main <= 6
    bursts = [(0, 128), (128, T - 128)] if T > 128 else [(0, T)]

    mesh = plsc.VectorSubcoreMesh(core_axis_name="c", subcore_axis_name="s")

    @functools.partial(
        pl.kernel,
        mesh=mesh,
        out_type=jax.ShapeDtypeStruct((BATCH, T, _D), jnp.float32),
        scratch_types=[
            pltpu.VMEM((sents_per_w, T), jnp.int32),
            pltpu.VMEM((_NBUF, _SENT, T, _D), jnp.float32),
            pltpu.SemaphoreType.DMA((_NBUF,)),
            pltpu.SemaphoreType.DMA((_NBUF,)),
        ],
        compiler_params=pltpu.CompilerParams(use_tc_tiling_on_sc=False),
    )
    def body(x_hbm, t_hbm, out_hbm, idx_v, bufs, sg, so):
        wid = lax.axis_index("s") * NC + lax.axis_index("c")
        sent_base = wid * sents_per_w

        def fire_gather(s, c):
            for q in range(_SENT):
                for (off, ln) in bursts:
                    pltpu.async_copy(
                        t_hbm.at[idx_v.at[c * _SENT + q, pl.ds(off, ln)]],
                        bufs.at[s, q, pl.ds(off, ln)],
                        sg.at[s],
                    )

        def drain(sem_row):
            pltpu.make_async_copy(
                out_hbm.at[pl.ds(0, _SENT)], bufs.at[0], sem_row
            ).wait()

        def scale(s):
            RU = 8

            def srow(q):
                def iter_(r, carry):
                    for u in range(RU):
                        t = r * RU + u
                        for j4 in range(_D // 16):
                            sl = pl.ds(16 * j4, 16)
                            bufs[s, q, t, sl] = bufs[s, q, t, sl] * _SCALE
                    return carry
                lax.fori_loop(0, T // RU, iter_, 0)
                for t in range(T - T % RU, T):
                    for j4 in range(_D // 16):
                        sl = pl.ds(16 * j4, 16)
                        bufs[s, q, t, sl] = bufs[s, q, t, sl] * _SCALE

            for q in range(_SENT):
                srow(q)

        def fire_out(s, c):
            sb = sent_base + c * _SENT
            pltpu.async_copy(bufs.at[s], out_hbm.at[pl.ds(sb, _SENT)], so.at[s])

        pltpu.sync_copy(x_hbm.at[pl.ds(sent_base, sents_per_w)], idx_v)
        fire_gather(0, 0)
        fire_gather(1, 1)

        drain(sg.at[0])
        scale(0)
        fire_gather(2, 2)
        fire_out(0, 0)

        drain(sg.at[1])
        scale(1)
        drain(so.at[0])
        fire_gather(0, 3)
        fire_out(1, 1)

        drain(sg.at[2])
        scale(2)
        drain(so.at[1])
        fire_gather(1, 4)
        fire_out(2, 2)

        def ring(g, carry):
            c0 = g * _NBUF
            for k in range(_NBUF):
                c = c0 + k
                drain(sg.at[k])
                scale(k)
                drain(so.at[(k + 2) % _NBUF])
                fire_gather((k + 2) % _NBUF, c + 2)
                fire_out(k, c)
            return carry

        lax.fori_loop(1, n_main // _NBUF, ring, 0)

        for c in range(n_main, n_chunks):
            s = c % _NBUF
            drain(sg.at[s])
            scale(s)
            drain(so.at[(s + 2) % _NBUF])
            if c + 2 < n_chunks:
                fire_gather((s + 2) % _NBUF, c + 2)
            fire_out(s, c)

        drain(so.at[(n_chunks - 1) % _NBUF])

    return body


def kernel(x, table):
    x32 = x.astype(jnp.int32)
    return _build(x.shape[0], x.shape[1], table.shape[0])(x32, table)
